# baseline (speedup 1.0000x reference)
import functools

import jax
import jax.numpy as jnp
from jax import lax
from jax.experimental import pallas as pl
from jax.experimental.pallas import tpu as pltpu

_B = 1024
_K = 100000
_BM = 16
_NBUF = 8
_NCH = _B // _BM


def _tc_body(v_ref, s_hbm, o_ref, sbuf, sem):
    def start(c, slot):
        pltpu.make_async_copy(
            s_hbm.at[pl.ds(c * _BM, _BM), :], sbuf.at[slot],
            sem.at[slot]).start()

    def wait(c, slot):
        pltpu.make_async_copy(
            s_hbm.at[pl.ds(c * _BM, _BM), :], sbuf.at[slot],
            sem.at[slot]).wait()

    for b in range(_NBUF):
        start(b, b)

    def ring_body(p, carry):
        for b in range(_NBUF):
            c = _NBUF * p + b
            wait(c, b)
            prod = sbuf[b] * v_ref[...]
            o_ref[pl.ds(c * _BM, _BM), :] = jnp.sum(
                prod, axis=1, keepdims=True)

            @pl.when(c + _NBUF < _NCH)
            def _(c=c, b=b):
                start(c + _NBUF, b)
        return carry

    lax.fori_loop(0, _NCH // _NBUF, ring_body, 0)


_tc_matvec = pl.pallas_call(
    _tc_body,
    grid=(),
    in_specs=[
        pl.BlockSpec(memory_space=pltpu.VMEM),
        pl.BlockSpec(memory_space=pltpu.HBM),
    ],
    out_specs=pl.BlockSpec(memory_space=pltpu.VMEM),
    out_shape=jax.ShapeDtypeStruct((_B, 1), jnp.float32),
    scratch_shapes=[
        pltpu.VMEM((_NBUF, _BM, _K), jnp.float32),
        pltpu.SemaphoreType.DMA((_NBUF,)),
    ],
)


def kernel(state, values):
    return _tc_matvec(values.reshape(1, _K), state)


# speedup vs baseline: 1.0074x; 1.0074x over previous
import jax
import jax.numpy as jnp
from jax.experimental import pallas as pl
from jax.experimental.pallas import tpu as pltpu

_B = 1024
_K = 100000
_BM = 64
_NM = _B // _BM


def _body(s_ref, v_ref, o_ref):
    o_ref[...] = jnp.sum(s_ref[...] * v_ref[...], axis=1, keepdims=True)


_matvec = pl.pallas_call(
    _body,
    grid=(_NM,),
    in_specs=[
        pl.BlockSpec((_BM, _K), lambda b: (b, 0)),
        pl.BlockSpec((1, _K), lambda b: (0, 0)),
    ],
    out_specs=pl.BlockSpec((_BM, 1), lambda b: (b, 0)),
    out_shape=jax.ShapeDtypeStruct((_B, 1), jnp.float32),
)


def kernel(state, values):
    return _matvec(state, values.reshape(1, _K))


# speedup vs baseline: 1.0097x; 1.0023x over previous
import jax
import jax.numpy as jnp
from jax.experimental import pallas as pl
from jax.experimental.pallas import tpu as pltpu

_B = 1024
_K = 100000
_BM = 64
_NM = _B // _BM


def _body(s_ref, v_ref, o_ref):
    o_ref[...] = jnp.sum(s_ref[...] * v_ref[...], axis=1, keepdims=True)


_matvec = pl.pallas_call(
    _body,
    grid=(_NM,),
    in_specs=[
        pl.BlockSpec((_BM, _K), lambda b: (b, 0)),
        pl.BlockSpec((1, _K), lambda b: (0, 0)),
    ],
    out_specs=pl.BlockSpec((_BM, 1), lambda b: (b, 0)),
    out_shape=jax.ShapeDtypeStruct((_B, 1), jnp.float32),
    compiler_params=pltpu.CompilerParams(
        dimension_semantics=("parallel",)),
)


def kernel(state, values):
    return _matvec(state, values.reshape(1, _K))
